# Initial kernel scaffold; baseline (speedup 1.0000x reference)
import jax
import jax.numpy as jnp
from jax.experimental import pallas as pl


def kernel(indices, table):
    raise NotImplementedError("write your pallas kernel here")


# speedup vs baseline: 4.6186x; 4.6186x over previous
import functools

import jax
import jax.numpy as jnp
from jax import lax
from jax.experimental import pallas as pl
from jax.experimental.pallas import tpu as pltpu
from jax.experimental.pallas import tpu_sc as plsc

VOCAB = 1000
EMB_DIM = 64
BATCH = 4096
HIST = 50

CHUNK = 128
TOTAL = BATCH * HIST
NCHUNKS = TOTAL // CHUNK


def _make_kernel():
    info = plsc.get_sparse_core_info()
    nc, ns = info.num_cores, info.num_subcores
    nw = nc * ns
    cpw = NCHUNKS // nw

    mesh = plsc.VectorSubcoreMesh(core_axis_name="c", subcore_axis_name="s")

    @functools.partial(
        pl.kernel,
        mesh=mesh,
        out_type=jax.ShapeDtypeStruct((NCHUNKS, CHUNK, EMB_DIM), jnp.float32),
        scratch_types=[
            pltpu.VMEM((cpw, CHUNK), jnp.int32),
            pltpu.VMEM((CHUNK, EMB_DIM), jnp.float32),
            pltpu.SemaphoreType.DMA,
        ],
        compiler_params=pltpu.CompilerParams(use_tc_tiling_on_sc=False),
    )
    def emb_kernel(idx_hbm, table_hbm, out_hbm, idx_v, rows_v, sem):
        wid = lax.axis_index("s") * nc + lax.axis_index("c")
        base = wid * cpw
        pltpu.sync_copy(idx_hbm.at[wid], idx_v)

        def body(j, carry):
            pltpu.async_copy(table_hbm.at[idx_v.at[j]], rows_v, sem).wait()
            pltpu.sync_copy(rows_v, out_hbm.at[base + j])
            return carry

        lax.fori_loop(0, cpw, body, 0)

    return emb_kernel, nw


_emb_kernel, _NW = _make_kernel()


def kernel(indices, table):
    idx3 = indices.reshape(_NW, NCHUNKS // _NW, CHUNK)
    out = _emb_kernel(idx3, table)
    return out.reshape(BATCH, HIST, EMB_DIM)


# speedup vs baseline: 4.7630x; 1.0313x over previous
import functools

import jax
import jax.numpy as jnp
from jax import lax
from jax.experimental import pallas as pl
from jax.experimental.pallas import tpu as pltpu
from jax.experimental.pallas import tpu_sc as plsc

VOCAB = 1000
EMB_DIM = 64
BATCH = 4096
HIST = 50

CHUNK = 128
TOTAL = BATCH * HIST
NCHUNKS = TOTAL // CHUNK

NBUF = 10
LAG = 5


def _make_kernel():
    info = plsc.get_sparse_core_info()
    nc, ns = info.num_cores, info.num_subcores
    nw = nc * ns
    cpw = NCHUNKS // nw
    nouter = cpw // NBUF

    mesh = plsc.VectorSubcoreMesh(core_axis_name="c", subcore_axis_name="s")

    scratch = (
        [pltpu.VMEM((cpw, CHUNK), jnp.int32)]
        + [pltpu.VMEM((CHUNK, EMB_DIM), jnp.float32) for _ in range(NBUF)]
        + [pltpu.SemaphoreType.DMA for _ in range(2 * NBUF)]
    )

    @functools.partial(
        pl.kernel,
        mesh=mesh,
        out_type=jax.ShapeDtypeStruct((NCHUNKS, CHUNK, EMB_DIM), jnp.float32),
        scratch_types=scratch,
        compiler_params=pltpu.CompilerParams(use_tc_tiling_on_sc=False),
    )
    def emb_kernel(idx_hbm, table_hbm, out_hbm, idx_v, *bufs_and_sems):
        bufs = bufs_and_sems[:NBUF]
        sem_g = bufs_and_sems[NBUF:2 * NBUF]
        sem_o = bufs_and_sems[2 * NBUF:]

        wid = lax.axis_index("s") * nc + lax.axis_index("c")
        base = wid * cpw
        pltpu.sync_copy(idx_hbm.at[wid], idx_v)

        def gather(chunk, b):
            pltpu.async_copy(table_hbm.at[idx_v.at[chunk]], bufs[b], sem_g[b])

        def put(chunk, b):
            pltpu.async_copy(bufs[b], out_hbm.at[base + chunk], sem_o[b])

        def wait_gather(b):
            pltpu.make_async_copy(
                table_hbm.at[idx_v.at[0]], bufs[b], sem_g[b]).wait()

        def wait_put(b):
            pltpu.make_async_copy(bufs[b], out_hbm.at[base], sem_o[b]).wait()

        for b in range(LAG):
            gather(b, b)

        def round_body(i, carry):
            for b in range(NBUF):
                j = i * NBUF + b
                k = j + LAG
                bk = (b + LAG) % NBUF

                @pl.when(k < cpw)
                def _():
                    @pl.when(k >= NBUF)
                    def _():
                        wait_put(bk)

                    gather(k, bk)

                wait_gather(b)
                put(j, b)
            return carry

        lax.fori_loop(0, nouter, round_body, 0)

        for b in range(NBUF):
            wait_put(b)

    return emb_kernel, nw


_emb_kernel, _NW = _make_kernel()


def kernel(indices, table):
    idx3 = indices.reshape(_NW, NCHUNKS // _NW, CHUNK)
    out = _emb_kernel(idx3, table)
    return out.reshape(BATCH, HIST, EMB_DIM)
